# Initial kernel scaffold; baseline (speedup 1.0000x reference)
import jax
import jax.numpy as jnp
from jax.experimental import pallas as pl


def kernel(x, tables):
    raise NotImplementedError("write your pallas kernel here")


# speedup vs baseline: 36.7501x; 36.7501x over previous
import functools

import jax
import jax.numpy as jnp
import numpy as np
from jax import lax
from jax.experimental import pallas as pl
from jax.experimental.pallas import tpu as pltpu
from jax.experimental.pallas import tpu_sc as plsc

_N_LEVELS = 16
_F = 2
_LOG2_T = 19
_T = 1 << _LOG2_T
_BASE = np.float32(16.0)
_FINEST = np.float32(512.0)
_B_GROWTH = np.float32(
    np.exp((np.log(_FINEST) - np.log(_BASE)) / np.float32(_N_LEVELS - 1)))
_RES = [np.float32(np.floor(_BASE * (_B_GROWTH ** i))) for i in range(_N_LEVELS)]
_TBL = [int((int(r) + 1) ** 3) if int(r) ** 3 < _T else _T for r in _RES]
_OFFS = [(i, j, k) for i in (0, 1) for j in (0, 1) for k in (0, 1)]
_P1 = int(np.uint32(2654435761).view(np.int32))
_P2 = int(np.uint32(805459861).view(np.int32))
_MASK = _T - 1
_N_DENSE = sum(1 for r in _RES if int(r) ** 3 < _T)

_LANES = 16
_NC = 2
_NS = 16
_NW = _NC * _NS
_OUTW = 2 * _N_LEVELS

_WOFF = np.concatenate([[0], np.cumsum([2 * s for s in _TBL])]).astype(np.int64)

_CST_F = np.repeat(np.array([float(r) * 0.5 for r in _RES], np.float32), _LANES)
_CST_I = np.concatenate([
    np.repeat(_WOFF[:_N_LEVELS].astype(np.int32), _LANES),
    np.repeat(np.array([int(r) for r in _RES], np.int32), _LANES),
])


def _sc_body(batch, chunk, x_hbm, tcat, cstf_hbm, csti_hbm, out_hbm,
             x_v, idx_v, rows_v, out_v, cstf_v, csti_v, sem):
  cid = lax.axis_index("c")
  sid = lax.axis_index("s")
  wid = sid * _NC + cid
  ppw = batch // _NW
  nchunks = ppw // chunk
  g8 = chunk // 8
  lane = lax.iota(jnp.int32, _LANES)
  par = lane & 1
  pc = (lane >> 1) * _OUTW + par

  pltpu.sync_copy(cstf_hbm, cstf_v)
  pltpu.sync_copy(csti_hbm, csti_v)

  def load_x(g):
    off = g * _LANES
    return (x_v[0, pl.ds(off, _LANES)],
            x_v[1, pl.ds(off, _LANES)],
            x_v[2, pl.ds(off, _LANES)])

  def lerp(g, carry, *, hrv, lvl):
    del carry
    x0, x1, x2 = load_x(g)
    t0 = (x0 + 1.0) * hrv
    t1 = (x1 + 1.0) * hrv
    t2 = (x2 + 1.0) * hrv
    w0 = t0 - t0.astype(jnp.int32).astype(jnp.float32)
    w1 = t1 - t1.astype(jnp.int32).astype(jnp.float32)
    w2 = t2 - t2.astype(jnp.int32).astype(jnp.float32)
    vb = g * 128
    v = [rows_v[pl.ds(vb + c * _LANES, _LANES)] for c in range(8)]
    c00 = v[0] + w0 * (v[4] - v[0])
    c01 = v[1] + w0 * (v[5] - v[1])
    c10 = v[2] + w0 * (v[6] - v[2])
    c11 = v[3] + w0 * (v[7] - v[3])
    c0 = c00 + w1 * (c10 - c00)
    c1 = c01 + w1 * (c11 - c01)
    res = c0 + w2 * (c1 - c0)
    plsc.store_scatter(out_v, [g * (8 * _OUTW) + pc + 2 * lvl], res)
    return 0

  def chunk_body(ci, _):
    base = wid * ppw + ci * chunk
    pltpu.sync_copy(x_hbm.at[:, pl.ds(2 * base, 2 * chunk)], x_v)

    def dense_level(lvl, _):
      hrv = cstf_v[pl.ds(lvl * _LANES, _LANES)]
      offv = csti_v[pl.ds(lvl * _LANES, _LANES)] + par
      riv = csti_v[pl.ds(_N_LEVELS * _LANES + lvl * _LANES, _LANES)]
      ri2v = riv * riv
      ccs = [2 * (i * ri2v + j * riv + k) for (i, j, k) in _OFFS]

      def idx_body(g, _):
        x0, x1, x2 = load_x(g)
        b0 = ((x0 + 1.0) * hrv).astype(jnp.int32)
        b1 = ((x1 + 1.0) * hrv).astype(jnp.int32)
        b2 = ((x2 + 1.0) * hrv).astype(jnp.int32)
        ib = g * 128
        bid = (b0 * ri2v + b1 * riv + b2) * 2 + offv
        for c in range(8):
          idx_v[pl.ds(ib + c * _LANES, _LANES)] = bid + ccs[c]
        return 0

      lax.fori_loop(0, g8, idx_body, 0)
      pltpu.async_copy(tcat.at[idx_v], rows_v, sem).wait()
      lax.fori_loop(0, g8, functools.partial(lerp, hrv=hrv, lvl=lvl), 0)
      return 0

    lax.fori_loop(0, _N_DENSE, dense_level, 0)

    def hash_level(lvl, _):
      hrv = cstf_v[pl.ds(lvl * _LANES, _LANES)]
      offv = csti_v[pl.ds(lvl * _LANES, _LANES)] + par

      def idx_body(g, _):
        x0, x1, x2 = load_x(g)
        b0 = ((x0 + 1.0) * hrv).astype(jnp.int32)
        b1 = ((x1 + 1.0) * hrv).astype(jnp.int32)
        b2 = ((x2 + 1.0) * hrv).astype(jnp.int32)
        ib = g * 128
        m1a = b1 * jnp.int32(_P1)
        m1b = m1a + jnp.int32(_P1)
        m2a = b2 * jnp.int32(_P2)
        m2b = m2a + jnp.int32(_P2)
        b0p = b0 + 1
        for c, (i, j, k) in enumerate(_OFFS):
          h = (b0p if i else b0) ^ (m1b if j else m1a) ^ (m2b if k else m2a)
          idx_v[pl.ds(ib + c * _LANES, _LANES)] = (
              ((h & jnp.int32(_MASK)) << 1) + offv)
        return 0

      lax.fori_loop(0, g8, idx_body, 0)
      pltpu.async_copy(tcat.at[idx_v], rows_v, sem).wait()
      lax.fori_loop(0, g8, functools.partial(lerp, hrv=hrv, lvl=lvl), 0)
      return 0

    lax.fori_loop(_N_DENSE, _N_LEVELS, hash_level, 0)

    pltpu.sync_copy(out_v, out_hbm.at[pl.ds(base * _OUTW, chunk * _OUTW)])
    return 0

  lax.fori_loop(0, nchunks, chunk_body, 0)


@functools.partial(jax.jit, static_argnames=("interpret",))
def _run(x, tables, interpret=False):
  batch = x.shape[0]
  chunk = min(1024, batch // _NW)
  mesh = plsc.VectorSubcoreMesh(
      core_axis_name="c", subcore_axis_name="s",
      num_cores=_NC, num_subcores=_NS)
  body = functools.partial(_sc_body, batch, chunk)
  fn = pl.kernel(
      body,
      out_type=jax.ShapeDtypeStruct((batch * _OUTW,), jnp.float32),
      mesh=mesh,
      scratch_types=[
          pltpu.VMEM((3, 2 * chunk), jnp.float32),
          pltpu.VMEM((16 * chunk,), jnp.int32),
          pltpu.VMEM((16 * chunk,), jnp.float32),
          pltpu.VMEM((chunk * _OUTW,), jnp.float32),
          pltpu.VMEM((_N_LEVELS * _LANES,), jnp.float32),
          pltpu.VMEM((2 * _N_LEVELS * _LANES,), jnp.int32),
          pltpu.SemaphoreType.DMA,
      ],
      compiler_params=pltpu.CompilerParams(needs_layout_passes=False),
      interpret=interpret,
  )
  xp = jnp.repeat(x.T, 2, axis=1)
  tcat = jnp.concatenate([t.reshape(-1) for t in tables])
  out = fn(xp, tcat, jnp.asarray(_CST_F), jnp.asarray(_CST_I))
  return out.reshape(batch, _OUTW)


def kernel(x, tables):
  return _run(x, tables)


# speedup vs baseline: 40.0803x; 1.0906x over previous
import functools

import jax
import jax.numpy as jnp
import numpy as np
from jax import lax
from jax.experimental import pallas as pl
from jax.experimental.pallas import tpu as pltpu
from jax.experimental.pallas import tpu_sc as plsc

_N_LEVELS = 16
_F = 2
_LOG2_T = 19
_T = 1 << _LOG2_T
_BASE = np.float32(16.0)
_FINEST = np.float32(512.0)
_B_GROWTH = np.float32(
    np.exp((np.log(_FINEST) - np.log(_BASE)) / np.float32(_N_LEVELS - 1)))
_RES = [np.float32(np.floor(_BASE * (_B_GROWTH ** i))) for i in range(_N_LEVELS)]
_TBL = [int((int(r) + 1) ** 3) if int(r) ** 3 < _T else _T for r in _RES]
_OFFS = [(i, j, k) for i in (0, 1) for j in (0, 1) for k in (0, 1)]
_P1 = int(np.uint32(2654435761).view(np.int32))
_P2 = int(np.uint32(805459861).view(np.int32))
_MASK = _T - 1
_N_DENSE = sum(1 for r in _RES if int(r) ** 3 < _T)

_LANES = 16
_NC = 2
_NS = 16
_NW = _NC * _NS
_OUTW = 2 * _N_LEVELS


def _sc_body(batch, chunk, *refs):
  x_hbm = refs[0]
  tabs = refs[1:1 + _N_LEVELS]
  out_hbm = refs[1 + _N_LEVELS]
  x_v, idx_v, rows_v, out_v, sem = refs[2 + _N_LEVELS:]

  cid = lax.axis_index("c")
  sid = lax.axis_index("s")
  wid = sid * _NC + cid
  ppw = batch // _NW
  nchunks = ppw // chunk
  g8 = chunk // 8
  lane = lax.iota(jnp.int32, _LANES)
  par = lane & 1
  pc = (lane >> 1) * _OUTW + par
  xg = (lane >> 1) * 3

  def load_x(g):
    b = g * 24 + xg
    return (plsc.load_gather(x_v, [b]),
            plsc.load_gather(x_v, [b + 1]),
            plsc.load_gather(x_v, [b + 2]))

  def lerp(g, carry, *, hr, lvl):
    del carry
    x0, x1, x2 = load_x(g)
    t0 = (x0 + 1.0) * hr
    t1 = (x1 + 1.0) * hr
    t2 = (x2 + 1.0) * hr
    w0 = t0 - t0.astype(jnp.int32).astype(jnp.float32)
    w1 = t1 - t1.astype(jnp.int32).astype(jnp.float32)
    w2 = t2 - t2.astype(jnp.int32).astype(jnp.float32)
    vb = g * 128
    v = [rows_v[pl.ds(vb + c * _LANES, _LANES)] for c in range(8)]
    c00 = v[0] + w0 * (v[4] - v[0])
    c01 = v[1] + w0 * (v[5] - v[1])
    c10 = v[2] + w0 * (v[6] - v[2])
    c11 = v[3] + w0 * (v[7] - v[3])
    c0 = c00 + w1 * (c10 - c00)
    c1 = c01 + w1 * (c11 - c01)
    res = c0 + w2 * (c1 - c0)
    plsc.store_scatter(out_v, [g * (8 * _OUTW) + pc + 2 * lvl], res)
    return 0

  def chunk_body(ci, _):
    base = wid * ppw + ci * chunk
    pltpu.sync_copy(x_hbm.at[pl.ds(3 * base, 3 * chunk)], x_v)

    for lvl in range(_N_DENSE):
      hr = float(_RES[lvl]) * 0.5
      ri = int(_RES[lvl])
      ri2 = ri * ri
      ccs = [2 * (i * ri2 + j * ri + k) for (i, j, k) in _OFFS]

      def idx_dense(g, _, hr=hr, ri=ri, ri2=ri2, ccs=ccs):
        x0, x1, x2 = load_x(g)
        b0 = ((x0 + 1.0) * hr).astype(jnp.int32)
        b1 = ((x1 + 1.0) * hr).astype(jnp.int32)
        b2 = ((x2 + 1.0) * hr).astype(jnp.int32)
        ib = g * 128
        bid = (b0 * ri2 + b1 * ri + b2) * 2 + par
        for c in range(8):
          idx_v[pl.ds(ib + c * _LANES, _LANES)] = bid + ccs[c]
        return 0

      lax.fori_loop(0, g8, idx_dense, 0)
      pltpu.async_copy(tabs[lvl].at[idx_v], rows_v, sem).wait()
      lax.fori_loop(0, g8, functools.partial(lerp, hr=hr, lvl=lvl), 0)

    for lvl in range(_N_DENSE, _N_LEVELS):
      hr = float(_RES[lvl]) * 0.5

      def idx_hash(g, _, hr=hr):
        x0, x1, x2 = load_x(g)
        b0 = ((x0 + 1.0) * hr).astype(jnp.int32)
        b1 = ((x1 + 1.0) * hr).astype(jnp.int32)
        b2 = ((x2 + 1.0) * hr).astype(jnp.int32)
        ib = g * 128
        m1a = b1 * jnp.int32(_P1)
        m1b = m1a + jnp.int32(_P1)
        m2a = b2 * jnp.int32(_P2)
        m2b = m2a + jnp.int32(_P2)
        b0p = b0 + 1
        for c, (i, j, k) in enumerate(_OFFS):
          h = (b0p if i else b0) ^ (m1b if j else m1a) ^ (m2b if k else m2a)
          idx_v[pl.ds(ib + c * _LANES, _LANES)] = (
              ((h & jnp.int32(_MASK)) << 1) + par)
        return 0

      lax.fori_loop(0, g8, idx_hash, 0)
      pltpu.async_copy(tabs[lvl].at[idx_v], rows_v, sem).wait()
      lax.fori_loop(0, g8, functools.partial(lerp, hr=hr, lvl=lvl), 0)

    pltpu.sync_copy(out_v, out_hbm.at[pl.ds(base * _OUTW, chunk * _OUTW)])
    return 0

  lax.fori_loop(0, nchunks, chunk_body, 0)


@functools.partial(jax.jit, static_argnames=("interpret",))
def _run(x, tables, interpret=False):
  batch = x.shape[0]
  chunk = min(1024, batch // _NW)
  mesh = plsc.VectorSubcoreMesh(
      core_axis_name="c", subcore_axis_name="s",
      num_cores=_NC, num_subcores=_NS)
  body = functools.partial(_sc_body, batch, chunk)
  fn = pl.kernel(
      body,
      out_type=jax.ShapeDtypeStruct((batch * _OUTW,), jnp.float32),
      mesh=mesh,
      scratch_types=[
          pltpu.VMEM((3 * chunk,), jnp.float32),
          pltpu.VMEM((16 * chunk,), jnp.int32),
          pltpu.VMEM((16 * chunk,), jnp.float32),
          pltpu.VMEM((chunk * _OUTW,), jnp.float32),
          pltpu.SemaphoreType.DMA,
      ],
      compiler_params=pltpu.CompilerParams(needs_layout_passes=False),
      interpret=interpret,
  )
  out = fn(x.reshape(-1), *[t.reshape(-1) for t in tables])
  return out.reshape(batch, _OUTW)


def kernel(x, tables):
  return _run(x, tables)


# speedup vs baseline: 40.5914x; 1.0128x over previous
import functools

import jax
import jax.numpy as jnp
import numpy as np
from jax import lax
from jax.experimental import pallas as pl
from jax.experimental.pallas import tpu as pltpu
from jax.experimental.pallas import tpu_sc as plsc

_N_LEVELS = 16
_F = 2
_LOG2_T = 19
_T = 1 << _LOG2_T
_BASE = np.float32(16.0)
_FINEST = np.float32(512.0)
_B_GROWTH = np.float32(
    np.exp((np.log(_FINEST) - np.log(_BASE)) / np.float32(_N_LEVELS - 1)))
_RES = [np.float32(np.floor(_BASE * (_B_GROWTH ** i))) for i in range(_N_LEVELS)]
_TBL = [int((int(r) + 1) ** 3) if int(r) ** 3 < _T else _T for r in _RES]
_OFFS = [(i, j, k) for i in (0, 1) for j in (0, 1) for k in (0, 1)]
_P1 = int(np.uint32(2654435761).view(np.int32))
_P2 = int(np.uint32(805459861).view(np.int32))
_MASK = _T - 1
_N_DENSE = sum(1 for r in _RES if int(r) ** 3 < _T)

_LANES = 16
_NC = 2
_NS = 16
_NW = _NC * _NS
_OUTW = 2 * _N_LEVELS


def _sc_body(batch, chunk, *refs):
  x_hbm = refs[0]
  tabs = refs[1:1 + _N_LEVELS]
  out_hbm = refs[1 + _N_LEVELS]
  x_v, idx_v, rows_v, out_v, sem = refs[2 + _N_LEVELS:]

  cid = lax.axis_index("c")
  sid = lax.axis_index("s")
  wid = sid * _NC + cid
  ppw = batch // _NW
  nchunks = ppw // chunk
  g16 = chunk // _LANES
  lane = lax.iota(jnp.int32, _LANES)
  x3 = lane * 3
  ow = lane * _OUTW

  def load_x(g):
    b = g * 48 + x3
    return (plsc.load_gather(x_v, [b]),
            plsc.load_gather(x_v, [b + 1]),
            plsc.load_gather(x_v, [b + 2]))

  def lerp(g, carry, *, hr, lvl):
    del carry
    x0, x1, x2 = load_x(g)
    t0 = (x0 + 1.0) * hr
    t1 = (x1 + 1.0) * hr
    t2 = (x2 + 1.0) * hr
    w0 = t0 - t0.astype(jnp.int32).astype(jnp.float32)
    w1 = t1 - t1.astype(jnp.int32).astype(jnp.float32)
    w2 = t2 - t2.astype(jnp.int32).astype(jnp.float32)
    rb = g * 256 + lane
    outs = []
    for f in (0, 1):
      v = [plsc.load_gather(rows_v, [rb + (c * 2 + f) * _LANES])
           for c in range(8)]
      c00 = v[0] + w0 * (v[4] - v[0])
      c01 = v[1] + w0 * (v[5] - v[1])
      c10 = v[2] + w0 * (v[6] - v[2])
      c11 = v[3] + w0 * (v[7] - v[3])
      c0 = c00 + w1 * (c10 - c00)
      c1 = c01 + w1 * (c11 - c01)
      outs.append(c0 + w2 * (c1 - c0))
    pos = g * (_LANES * _OUTW) + ow + 2 * lvl
    plsc.store_scatter(out_v, [pos], outs[0])
    plsc.store_scatter(out_v, [pos + 1], outs[1])
    return 0

  def chunk_body(ci, _):
    base = wid * ppw + ci * chunk
    pltpu.sync_copy(x_hbm.at[pl.ds(3 * base, 3 * chunk)], x_v)

    for lvl in range(_N_DENSE):
      hr = float(_RES[lvl]) * 0.5
      ri = int(_RES[lvl])
      ri2 = ri * ri
      ccs = [i * ri2 + j * ri + k for (i, j, k) in _OFFS]

      def idx_dense(g, _, hr=hr, ri=ri, ri2=ri2, ccs=ccs):
        x0, x1, x2 = load_x(g)
        b0 = ((x0 + 1.0) * hr).astype(jnp.int32)
        b1 = ((x1 + 1.0) * hr).astype(jnp.int32)
        b2 = ((x2 + 1.0) * hr).astype(jnp.int32)
        ib = g * 256
        bid = b0 * ri2 + b1 * ri + b2
        for c in range(8):
          flat = 2 * (bid + ccs[c])
          idx_v[pl.ds(ib + 2 * c * _LANES, _LANES)] = flat
          idx_v[pl.ds(ib + (2 * c + 1) * _LANES, _LANES)] = flat + 1
        return 0

      lax.fori_loop(0, g16, idx_dense, 0)
      pltpu.async_copy(tabs[lvl].at[idx_v], rows_v, sem).wait()
      lax.fori_loop(0, g16, functools.partial(lerp, hr=hr, lvl=lvl), 0)

    for lvl in range(_N_DENSE, _N_LEVELS):
      hr = float(_RES[lvl]) * 0.5

      def idx_hash(g, _, hr=hr):
        x0, x1, x2 = load_x(g)
        b0 = ((x0 + 1.0) * hr).astype(jnp.int32)
        b1 = ((x1 + 1.0) * hr).astype(jnp.int32)
        b2 = ((x2 + 1.0) * hr).astype(jnp.int32)
        ib = g * 256
        m1a = b1 * jnp.int32(_P1)
        m1b = m1a + jnp.int32(_P1)
        m2a = b2 * jnp.int32(_P2)
        m2b = m2a + jnp.int32(_P2)
        b0p = b0 + 1
        for c, (i, j, k) in enumerate(_OFFS):
          h = (b0p if i else b0) ^ (m1b if j else m1a) ^ (m2b if k else m2a)
          flat = 2 * (h & jnp.int32(_MASK))
          idx_v[pl.ds(ib + 2 * c * _LANES, _LANES)] = flat
          idx_v[pl.ds(ib + (2 * c + 1) * _LANES, _LANES)] = flat + 1
        return 0

      lax.fori_loop(0, g16, idx_hash, 0)
      pltpu.async_copy(tabs[lvl].at[idx_v], rows_v, sem).wait()
      lax.fori_loop(0, g16, functools.partial(lerp, hr=hr, lvl=lvl), 0)

    pltpu.sync_copy(out_v, out_hbm.at[pl.ds(base * _OUTW, chunk * _OUTW)])
    return 0

  lax.fori_loop(0, nchunks, chunk_body, 0)


@functools.partial(jax.jit, static_argnames=("interpret",))
def _run(x, tables, interpret=False):
  batch = x.shape[0]
  chunk = min(1024, batch // _NW)
  mesh = plsc.VectorSubcoreMesh(
      core_axis_name="c", subcore_axis_name="s",
      num_cores=_NC, num_subcores=_NS)
  body = functools.partial(_sc_body, batch, chunk)
  fn = pl.kernel(
      body,
      out_type=jax.ShapeDtypeStruct((batch * _OUTW,), jnp.float32),
      mesh=mesh,
      scratch_types=[
          pltpu.VMEM((3 * chunk,), jnp.float32),
          pltpu.VMEM((16 * chunk,), jnp.int32),
          pltpu.VMEM((16 * chunk,), jnp.float32),
          pltpu.VMEM((chunk * _OUTW,), jnp.float32),
          pltpu.SemaphoreType.DMA,
      ],
      compiler_params=pltpu.CompilerParams(needs_layout_passes=False),
      interpret=interpret,
  )
  out = fn(x.reshape(-1), *[t.reshape(-1) for t in tables])
  return out.reshape(batch, _OUTW)


def kernel(x, tables):
  return _run(x, tables)


# speedup vs baseline: 43.2979x; 1.0667x over previous
import functools

import jax
import jax.numpy as jnp
import numpy as np
from jax import lax
from jax.experimental import pallas as pl
from jax.experimental.pallas import tpu as pltpu
from jax.experimental.pallas import tpu_sc as plsc

_N_LEVELS = 16
_F = 2
_LOG2_T = 19
_T = 1 << _LOG2_T
_BASE = np.float32(16.0)
_FINEST = np.float32(512.0)
_B_GROWTH = np.float32(
    np.exp((np.log(_FINEST) - np.log(_BASE)) / np.float32(_N_LEVELS - 1)))
_RES = [np.float32(np.floor(_BASE * (_B_GROWTH ** i))) for i in range(_N_LEVELS)]
_TBL = [int((int(r) + 1) ** 3) if int(r) ** 3 < _T else _T for r in _RES]
_OFFS = [(i, j, k) for i in (0, 1) for j in (0, 1) for k in (0, 1)]
_P1 = int(np.uint32(2654435761).view(np.int32))
_P2 = int(np.uint32(805459861).view(np.int32))
_MASK = _T - 1
_N_DENSE = sum(1 for r in _RES if int(r) ** 3 < _T)

_LANES = 16
_NC = 2
_NS = 16
_NW = _NC * _NS
_OUTW = 2 * _N_LEVELS


def _sc_body(batch, chunk, *refs):
  x_hbm = refs[0]
  tabs = refs[1:1 + _N_LEVELS]
  out_hbm = refs[1 + _N_LEVELS]
  (x_v, idx_v0, idx_v1, rows_v0, rows_v1, out_v,
   sem0, sem1) = refs[2 + _N_LEVELS:]
  idx_bufs = (idx_v0, idx_v1)
  rows_bufs = (rows_v0, rows_v1)
  sems = (sem0, sem1)

  cid = lax.axis_index("c")
  sid = lax.axis_index("s")
  wid = sid * _NC + cid
  ppw = batch // _NW
  nchunks = ppw // chunk
  g16 = chunk // _LANES
  lane = lax.iota(jnp.int32, _LANES)
  x3 = lane * 3
  ow = lane * _OUTW

  def load_x(g):
    b = g * 48 + x3
    return (plsc.load_gather(x_v, [b]),
            plsc.load_gather(x_v, [b + 1]),
            plsc.load_gather(x_v, [b + 2]))

  def lerp(g, carry, *, hr, lvl, rows_v):
    del carry
    x0, x1, x2 = load_x(g)
    t0 = (x0 + 1.0) * hr
    t1 = (x1 + 1.0) * hr
    t2 = (x2 + 1.0) * hr
    w0 = t0 - t0.astype(jnp.int32).astype(jnp.float32)
    w1 = t1 - t1.astype(jnp.int32).astype(jnp.float32)
    w2 = t2 - t2.astype(jnp.int32).astype(jnp.float32)
    rb = g * 256 + lane
    outs = []
    for f in (0, 1):
      v = [plsc.load_gather(rows_v, [rb + (c * 2 + f) * _LANES])
           for c in range(8)]
      c00 = v[0] + w0 * (v[4] - v[0])
      c01 = v[1] + w0 * (v[5] - v[1])
      c10 = v[2] + w0 * (v[6] - v[2])
      c11 = v[3] + w0 * (v[7] - v[3])
      c0 = c00 + w1 * (c10 - c00)
      c1 = c01 + w1 * (c11 - c01)
      outs.append(c0 + w2 * (c1 - c0))
    pos = g * (_LANES * _OUTW) + ow + 2 * lvl
    plsc.store_scatter(out_v, [pos], outs[0])
    plsc.store_scatter(out_v, [pos + 1], outs[1])
    return 0

  def make_idx_fn(lvl, idx_v):
    hr = float(_RES[lvl]) * 0.5
    if lvl < _N_DENSE:
      ri = int(_RES[lvl])
      ri2 = ri * ri
      ccs = [i * ri2 + j * ri + k for (i, j, k) in _OFFS]

      def idx_dense(g, _):
        x0, x1, x2 = load_x(g)
        b0 = ((x0 + 1.0) * hr).astype(jnp.int32)
        b1 = ((x1 + 1.0) * hr).astype(jnp.int32)
        b2 = ((x2 + 1.0) * hr).astype(jnp.int32)
        ib = g * 256
        bid = b0 * ri2 + b1 * ri + b2
        for c in range(8):
          flat = 2 * (bid + ccs[c])
          idx_v[pl.ds(ib + 2 * c * _LANES, _LANES)] = flat
          idx_v[pl.ds(ib + (2 * c + 1) * _LANES, _LANES)] = flat + 1
        return 0

      return idx_dense

    def idx_hash(g, _):
      x0, x1, x2 = load_x(g)
      b0 = ((x0 + 1.0) * hr).astype(jnp.int32)
      b1 = ((x1 + 1.0) * hr).astype(jnp.int32)
      b2 = ((x2 + 1.0) * hr).astype(jnp.int32)
      ib = g * 256
      m1a = b1 * jnp.int32(_P1)
      m1b = m1a + jnp.int32(_P1)
      m2a = b2 * jnp.int32(_P2)
      m2b = m2a + jnp.int32(_P2)
      b0p = b0 + 1
      for c, (i, j, k) in enumerate(_OFFS):
        h = (b0p if i else b0) ^ (m1b if j else m1a) ^ (m2b if k else m2a)
        flat = 2 * (h & jnp.int32(_MASK))
        idx_v[pl.ds(ib + 2 * c * _LANES, _LANES)] = flat
        idx_v[pl.ds(ib + (2 * c + 1) * _LANES, _LANES)] = flat + 1
      return 0

    return idx_hash

  def chunk_body(ci, _):
    base = wid * ppw + ci * chunk
    pltpu.sync_copy(x_hbm.at[pl.ds(3 * base, 3 * chunk)], x_v)

    lax.fori_loop(0, g16, make_idx_fn(0, idx_bufs[0]), 0)
    h_prev = pltpu.async_copy(tabs[0].at[idx_bufs[0]], rows_bufs[0], sems[0])
    for lvl in range(1, _N_LEVELS):
      b = lvl % 2
      lax.fori_loop(0, g16, make_idx_fn(lvl, idx_bufs[b]), 0)
      h = pltpu.async_copy(tabs[lvl].at[idx_bufs[b]], rows_bufs[b], sems[b])
      h_prev.wait()
      lax.fori_loop(
          0, g16,
          functools.partial(lerp, hr=float(_RES[lvl - 1]) * 0.5,
                            lvl=lvl - 1, rows_v=rows_bufs[1 - b]), 0)
      h_prev = h
    h_prev.wait()
    lax.fori_loop(
        0, g16,
        functools.partial(lerp, hr=float(_RES[_N_LEVELS - 1]) * 0.5,
                          lvl=_N_LEVELS - 1,
                          rows_v=rows_bufs[(_N_LEVELS - 1) % 2]), 0)

    pltpu.sync_copy(out_v, out_hbm.at[pl.ds(base * _OUTW, chunk * _OUTW)])
    return 0

  lax.fori_loop(0, nchunks, chunk_body, 0)


@functools.partial(jax.jit, static_argnames=("interpret",))
def _run(x, tables, interpret=False):
  batch = x.shape[0]
  chunk = min(1024, batch // _NW)
  mesh = plsc.VectorSubcoreMesh(
      core_axis_name="c", subcore_axis_name="s",
      num_cores=_NC, num_subcores=_NS)
  body = functools.partial(_sc_body, batch, chunk)
  fn = pl.kernel(
      body,
      out_type=jax.ShapeDtypeStruct((batch * _OUTW,), jnp.float32),
      mesh=mesh,
      scratch_types=[
          pltpu.VMEM((3 * chunk,), jnp.float32),
          pltpu.VMEM((16 * chunk,), jnp.int32),
          pltpu.VMEM((16 * chunk,), jnp.int32),
          pltpu.VMEM((16 * chunk,), jnp.float32),
          pltpu.VMEM((16 * chunk,), jnp.float32),
          pltpu.VMEM((chunk * _OUTW,), jnp.float32),
          pltpu.SemaphoreType.DMA,
          pltpu.SemaphoreType.DMA,
      ],
      compiler_params=pltpu.CompilerParams(needs_layout_passes=False),
      interpret=interpret,
  )
  out = fn(x.reshape(-1), *[t.reshape(-1) for t in tables])
  return out.reshape(batch, _OUTW)


def kernel(x, tables):
  return _run(x, tables)


# speedup vs baseline: 45.6549x; 1.0544x over previous
import functools

import jax
import jax.numpy as jnp
import numpy as np
from jax import lax
from jax.experimental import pallas as pl
from jax.experimental.pallas import tpu as pltpu
from jax.experimental.pallas import tpu_sc as plsc

_N_LEVELS = 16
_F = 2
_LOG2_T = 19
_T = 1 << _LOG2_T
_BASE = np.float32(16.0)
_FINEST = np.float32(512.0)
_B_GROWTH = np.float32(
    np.exp((np.log(_FINEST) - np.log(_BASE)) / np.float32(_N_LEVELS - 1)))
_RES = [np.float32(np.floor(_BASE * (_B_GROWTH ** i))) for i in range(_N_LEVELS)]
_TBL = [int((int(r) + 1) ** 3) if int(r) ** 3 < _T else _T for r in _RES]
_OFFS = [(i, j, k) for i in (0, 1) for j in (0, 1) for k in (0, 1)]
_P1 = int(np.uint32(2654435761).view(np.int32))
_P2 = int(np.uint32(805459861).view(np.int32))
_MASK = _T - 1
_N_DENSE = sum(1 for r in _RES if int(r) ** 3 < _T)

_LANES = 16
_NC = 2
_NS = 16
_NW = _NC * _NS
_OUTW = 2 * _N_LEVELS


def _sc_body(batch, chunk, *refs):
  x_hbm = refs[0]
  tabs = refs[1:1 + _N_LEVELS]
  out_hbm = refs[1 + _N_LEVELS]
  (x_v, idx_v0, idx_v1, rows_v0, rows_v1, out_v,
   sem0, sem1) = refs[2 + _N_LEVELS:]
  idx_bufs = (idx_v0, idx_v1)
  rows_bufs = (rows_v0, rows_v1)
  sems = (sem0, sem1)

  cid = lax.axis_index("c")
  sid = lax.axis_index("s")
  wid = sid * _NC + cid
  ppw = batch // _NW
  nchunks = ppw // chunk
  g16 = chunk // _LANES
  lane = lax.iota(jnp.int32, _LANES)
  x3 = lane * 3
  ow = lane * _OUTW

  def load_x(g):
    b = g * 48 + x3
    return (plsc.load_gather(x_v, [b]),
            plsc.load_gather(x_v, [b + 1]),
            plsc.load_gather(x_v, [b + 2]))

  def lerp(g, carry, *, hr, lvl, rows_v):
    del carry
    x0, x1, x2 = load_x(g)
    t0 = (x0 + 1.0) * hr
    t1 = (x1 + 1.0) * hr
    t2 = (x2 + 1.0) * hr
    w0 = t0 - t0.astype(jnp.int32).astype(jnp.float32)
    w1 = t1 - t1.astype(jnp.int32).astype(jnp.float32)
    w2 = t2 - t2.astype(jnp.int32).astype(jnp.float32)
    rb = g * 256 + lane
    outs = []
    for f in (0, 1):
      v = [plsc.load_gather(rows_v, [rb + (c * 2 + f) * _LANES])
           for c in range(8)]
      c00 = v[0] + w0 * (v[4] - v[0])
      c01 = v[1] + w0 * (v[5] - v[1])
      c10 = v[2] + w0 * (v[6] - v[2])
      c11 = v[3] + w0 * (v[7] - v[3])
      c0 = c00 + w1 * (c10 - c00)
      c1 = c01 + w1 * (c11 - c01)
      outs.append(c0 + w2 * (c1 - c0))
    pos = g * (_LANES * _OUTW) + ow + 2 * lvl
    plsc.store_scatter(out_v, [pos], outs[0])
    plsc.store_scatter(out_v, [pos + 1], outs[1])
    return 0

  def make_idx_fn(lvl, idx_v):
    hr = float(_RES[lvl]) * 0.5
    if lvl < _N_DENSE:
      ri = int(_RES[lvl])
      ri2 = ri * ri
      ccs = [i * ri2 + j * ri + k for (i, j, k) in _OFFS]

      def idx_dense(g, _):
        x0, x1, x2 = load_x(g)
        b0 = ((x0 + 1.0) * hr).astype(jnp.int32)
        b1 = ((x1 + 1.0) * hr).astype(jnp.int32)
        b2 = ((x2 + 1.0) * hr).astype(jnp.int32)
        ib = g * 256
        bid = b0 * ri2 + b1 * ri + b2
        for c in range(8):
          flat = 2 * (bid + ccs[c])
          idx_v[pl.ds(ib + 2 * c * _LANES, _LANES)] = flat
          idx_v[pl.ds(ib + (2 * c + 1) * _LANES, _LANES)] = flat + 1
        return 0

      return idx_dense

    def idx_hash(g, _):
      x0, x1, x2 = load_x(g)
      b0 = ((x0 + 1.0) * hr).astype(jnp.int32)
      b1 = ((x1 + 1.0) * hr).astype(jnp.int32)
      b2 = ((x2 + 1.0) * hr).astype(jnp.int32)
      ib = g * 256
      m1a = b1 * jnp.int32(_P1)
      m1b = m1a + jnp.int32(_P1)
      m2a = b2 * jnp.int32(_P2)
      m2b = m2a + jnp.int32(_P2)
      b0p = b0 + 1
      for c, (i, j, k) in enumerate(_OFFS):
        h = (b0p if i else b0) ^ (m1b if j else m1a) ^ (m2b if k else m2a)
        flat = 2 * (h & jnp.int32(_MASK))
        idx_v[pl.ds(ib + 2 * c * _LANES, _LANES)] = flat
        idx_v[pl.ds(ib + (2 * c + 1) * _LANES, _LANES)] = flat + 1
      return 0

    return idx_hash

  def chunk_body(ci, _):
    base = wid * ppw + ci * chunk
    pltpu.sync_copy(x_hbm.at[pl.ds(3 * base, 3 * chunk)], x_v)

    lax.fori_loop(0, g16, make_idx_fn(0, idx_bufs[0]), 0)
    h_prev = pltpu.async_copy(tabs[0].at[idx_bufs[0]], rows_bufs[0], sems[0])
    for lvl in range(1, _N_LEVELS):
      b = lvl % 2
      lax.fori_loop(0, g16, make_idx_fn(lvl, idx_bufs[b]), 0)
      h = pltpu.async_copy(tabs[lvl].at[idx_bufs[b]], rows_bufs[b], sems[b])
      h_prev.wait()
      lax.fori_loop(
          0, g16,
          functools.partial(lerp, hr=float(_RES[lvl - 1]) * 0.5,
                            lvl=lvl - 1, rows_v=rows_bufs[1 - b]), 0)
      h_prev = h
    h_prev.wait()
    lax.fori_loop(
        0, g16,
        functools.partial(lerp, hr=float(_RES[_N_LEVELS - 1]) * 0.5,
                          lvl=_N_LEVELS - 1,
                          rows_v=rows_bufs[(_N_LEVELS - 1) % 2]), 0)

    pltpu.sync_copy(out_v, out_hbm.at[pl.ds(base * _OUTW, chunk * _OUTW)])
    return 0

  lax.fori_loop(0, nchunks, chunk_body, 0)


@functools.partial(jax.jit, static_argnames=("interpret",))
def _run(x, tables, interpret=False):
  batch = x.shape[0]
  chunk = min(512, batch // _NW)
  mesh = plsc.VectorSubcoreMesh(
      core_axis_name="c", subcore_axis_name="s",
      num_cores=_NC, num_subcores=_NS)
  body = functools.partial(_sc_body, batch, chunk)
  fn = pl.kernel(
      body,
      out_type=jax.ShapeDtypeStruct((batch * _OUTW,), jnp.float32),
      mesh=mesh,
      scratch_types=[
          pltpu.VMEM((3 * chunk,), jnp.float32),
          pltpu.VMEM((16 * chunk,), jnp.int32),
          pltpu.VMEM((16 * chunk,), jnp.int32),
          pltpu.VMEM((16 * chunk,), jnp.float32),
          pltpu.VMEM((16 * chunk,), jnp.float32),
          pltpu.VMEM((chunk * _OUTW,), jnp.float32),
          pltpu.SemaphoreType.DMA,
          pltpu.SemaphoreType.DMA,
      ],
      compiler_params=pltpu.CompilerParams(needs_layout_passes=False),
      interpret=interpret,
  )
  out = fn(x.reshape(-1), *[t.reshape(-1) for t in tables])
  return out.reshape(batch, _OUTW)


def kernel(x, tables):
  return _run(x, tables)


# speedup vs baseline: 46.5279x; 1.0191x over previous
import functools

import jax
import jax.numpy as jnp
import numpy as np
from jax import lax
from jax.experimental import pallas as pl
from jax.experimental.pallas import tpu as pltpu
from jax.experimental.pallas import tpu_sc as plsc

_N_LEVELS = 16
_F = 2
_LOG2_T = 19
_T = 1 << _LOG2_T
_BASE = np.float32(16.0)
_FINEST = np.float32(512.0)
_B_GROWTH = np.float32(
    np.exp((np.log(_FINEST) - np.log(_BASE)) / np.float32(_N_LEVELS - 1)))
_RES = [np.float32(np.floor(_BASE * (_B_GROWTH ** i))) for i in range(_N_LEVELS)]
_TBL = [int((int(r) + 1) ** 3) if int(r) ** 3 < _T else _T for r in _RES]
_OFFS = [(i, j, k) for i in (0, 1) for j in (0, 1) for k in (0, 1)]
_P1 = int(np.uint32(2654435761).view(np.int32))
_P2 = int(np.uint32(805459861).view(np.int32))
_MASK = _T - 1
_N_DENSE = sum(1 for r in _RES if int(r) ** 3 < _T)

_LANES = 16
_NC = 2
_NS = 16
_NW = _NC * _NS
_OUTW = 2 * _N_LEVELS


def _sc_body(batch, chunk, *refs):
  x_hbm = refs[0]
  tabs = refs[1:1 + _N_LEVELS]
  out_hbm = refs[1 + _N_LEVELS]
  (x_v, idx_v0, idx_v1, rows_v0, rows_v1, out_v,
   sem0, sem1) = refs[2 + _N_LEVELS:]
  idx_bufs = (idx_v0, idx_v1)
  rows_bufs = (rows_v0, rows_v1)
  sems = (sem0, sem1)

  cid = lax.axis_index("c")
  sid = lax.axis_index("s")
  wid = sid * _NC + cid
  ppw = batch // _NW
  nchunks = ppw // chunk
  g16 = chunk // _LANES
  lane = lax.iota(jnp.int32, _LANES)
  x3 = lane * 3
  ow = lane * _OUTW

  def load_x(g):
    b = g * 48 + x3
    return (plsc.load_gather(x_v, [b]),
            plsc.load_gather(x_v, [b + 1]),
            plsc.load_gather(x_v, [b + 2]))

  def lerp(g, carry, *, hr, lvl, rows_v):
    del carry
    x0, x1, x2 = load_x(g)
    t0 = (x0 + 1.0) * hr
    t1 = (x1 + 1.0) * hr
    t2 = (x2 + 1.0) * hr
    w0 = t0 - t0.astype(jnp.int32).astype(jnp.float32)
    w1 = t1 - t1.astype(jnp.int32).astype(jnp.float32)
    w2 = t2 - t2.astype(jnp.int32).astype(jnp.float32)
    rb = g * 256 + lane
    outs = []
    for f in (0, 1):
      v = [plsc.load_gather(rows_v, [rb + (c * 2 + f) * _LANES])
           for c in range(8)]
      c00 = v[0] + w0 * (v[4] - v[0])
      c01 = v[1] + w0 * (v[5] - v[1])
      c10 = v[2] + w0 * (v[6] - v[2])
      c11 = v[3] + w0 * (v[7] - v[3])
      c0 = c00 + w1 * (c10 - c00)
      c1 = c01 + w1 * (c11 - c01)
      outs.append(c0 + w2 * (c1 - c0))
    pos = g * (_LANES * _OUTW) + ow + 2 * lvl
    plsc.store_scatter(out_v, [pos], outs[0])
    plsc.store_scatter(out_v, [pos + 1], outs[1])
    return 0

  def make_idx_fn(lvl, idx_v):
    hr = float(_RES[lvl]) * 0.5
    if lvl < _N_DENSE:
      ri = int(_RES[lvl])
      ri2 = ri * ri
      ccs = [i * ri2 + j * ri + k for (i, j, k) in _OFFS]

      def idx_dense(g, _):
        x0, x1, x2 = load_x(g)
        b0 = ((x0 + 1.0) * hr).astype(jnp.int32)
        b1 = ((x1 + 1.0) * hr).astype(jnp.int32)
        b2 = ((x2 + 1.0) * hr).astype(jnp.int32)
        ib = g * 256
        bid = b0 * ri2 + b1 * ri + b2
        for c in range(8):
          flat = 2 * (bid + ccs[c])
          idx_v[pl.ds(ib + 2 * c * _LANES, _LANES)] = flat
          idx_v[pl.ds(ib + (2 * c + 1) * _LANES, _LANES)] = flat + 1
        return 0

      return idx_dense

    def idx_hash(g, _):
      x0, x1, x2 = load_x(g)
      b0 = ((x0 + 1.0) * hr).astype(jnp.int32)
      b1 = ((x1 + 1.0) * hr).astype(jnp.int32)
      b2 = ((x2 + 1.0) * hr).astype(jnp.int32)
      ib = g * 256
      m1a = b1 * jnp.int32(_P1)
      m1b = m1a + jnp.int32(_P1)
      m2a = b2 * jnp.int32(_P2)
      m2b = m2a + jnp.int32(_P2)
      b0p = b0 + 1
      for c, (i, j, k) in enumerate(_OFFS):
        h = (b0p if i else b0) ^ (m1b if j else m1a) ^ (m2b if k else m2a)
        flat = 2 * (h & jnp.int32(_MASK))
        idx_v[pl.ds(ib + 2 * c * _LANES, _LANES)] = flat
        idx_v[pl.ds(ib + (2 * c + 1) * _LANES, _LANES)] = flat + 1
      return 0

    return idx_hash

  def chunk_body(ci, _):
    base = wid * ppw + ci * chunk
    pltpu.sync_copy(x_hbm.at[pl.ds(3 * base, 3 * chunk)], x_v)

    lax.fori_loop(0, g16, make_idx_fn(0, idx_bufs[0]), 0)
    h_prev = pltpu.async_copy(tabs[0].at[idx_bufs[0]], rows_bufs[0], sems[0])
    for lvl in range(1, _N_LEVELS):
      b = lvl % 2
      lax.fori_loop(0, g16, make_idx_fn(lvl, idx_bufs[b]), 0)
      h = pltpu.async_copy(tabs[lvl].at[idx_bufs[b]], rows_bufs[b], sems[b])
      h_prev.wait()
      lax.fori_loop(
          0, g16,
          functools.partial(lerp, hr=float(_RES[lvl - 1]) * 0.5,
                            lvl=lvl - 1, rows_v=rows_bufs[1 - b]), 0)
      h_prev = h
    h_prev.wait()
    lax.fori_loop(
        0, g16,
        functools.partial(lerp, hr=float(_RES[_N_LEVELS - 1]) * 0.5,
                          lvl=_N_LEVELS - 1,
                          rows_v=rows_bufs[(_N_LEVELS - 1) % 2]), 0)

    pltpu.sync_copy(out_v, out_hbm.at[pl.ds(base * _OUTW, chunk * _OUTW)])
    return 0

  lax.fori_loop(0, nchunks, chunk_body, 0)


@functools.partial(jax.jit, static_argnames=("interpret",))
def _run(x, tables, interpret=False):
  batch = x.shape[0]
  chunk = min(256, batch // _NW)
  mesh = plsc.VectorSubcoreMesh(
      core_axis_name="c", subcore_axis_name="s",
      num_cores=_NC, num_subcores=_NS)
  body = functools.partial(_sc_body, batch, chunk)
  fn = pl.kernel(
      body,
      out_type=jax.ShapeDtypeStruct((batch * _OUTW,), jnp.float32),
      mesh=mesh,
      scratch_types=[
          pltpu.VMEM((3 * chunk,), jnp.float32),
          pltpu.VMEM((16 * chunk,), jnp.int32),
          pltpu.VMEM((16 * chunk,), jnp.int32),
          pltpu.VMEM((16 * chunk,), jnp.float32),
          pltpu.VMEM((16 * chunk,), jnp.float32),
          pltpu.VMEM((chunk * _OUTW,), jnp.float32),
          pltpu.SemaphoreType.DMA,
          pltpu.SemaphoreType.DMA,
      ],
      compiler_params=pltpu.CompilerParams(needs_layout_passes=False),
      interpret=interpret,
  )
  out = fn(x.reshape(-1), *[t.reshape(-1) for t in tables])
  return out.reshape(batch, _OUTW)


def kernel(x, tables):
  return _run(x, tables)


# speedup vs baseline: 46.8271x; 1.0064x over previous
import functools

import jax
import jax.numpy as jnp
import numpy as np
from jax import lax
from jax.experimental import pallas as pl
from jax.experimental.pallas import tpu as pltpu
from jax.experimental.pallas import tpu_sc as plsc

_N_LEVELS = 16
_F = 2
_LOG2_T = 19
_T = 1 << _LOG2_T
_BASE = np.float32(16.0)
_FINEST = np.float32(512.0)
_B_GROWTH = np.float32(
    np.exp((np.log(_FINEST) - np.log(_BASE)) / np.float32(_N_LEVELS - 1)))
_RES = [np.float32(np.floor(_BASE * (_B_GROWTH ** i))) for i in range(_N_LEVELS)]
_TBL = [int((int(r) + 1) ** 3) if int(r) ** 3 < _T else _T for r in _RES]
_OFFS = [(i, j, k) for i in (0, 1) for j in (0, 1) for k in (0, 1)]
_P1 = int(np.uint32(2654435761).view(np.int32))
_P2 = int(np.uint32(805459861).view(np.int32))
_MASK = _T - 1
_N_DENSE = sum(1 for r in _RES if int(r) ** 3 < _T)

_LANES = 16
_NC = 2
_NS = 16
_NW = _NC * _NS
_OUTW = 2 * _N_LEVELS


def _sc_body(batch, chunk, *refs):
  x_hbm = refs[0]
  tabs = refs[1:1 + _N_LEVELS]
  out_hbm = refs[1 + _N_LEVELS]
  (x_v, idx_v0, idx_v1, rows_v0, rows_v1, out_v,
   sem0, sem1) = refs[2 + _N_LEVELS:]
  idx_bufs = (idx_v0, idx_v1)
  rows_bufs = (rows_v0, rows_v1)
  sems = (sem0, sem1)

  cid = lax.axis_index("c")
  sid = lax.axis_index("s")
  wid = sid * _NC + cid
  ppw = batch // _NW
  nchunks = ppw // chunk
  g16 = chunk // _LANES
  lane = lax.iota(jnp.int32, _LANES)
  x3 = lane * 3
  ow = lane * _OUTW

  def load_x(g):
    b = g * 48 + x3
    return (plsc.load_gather(x_v, [b]),
            plsc.load_gather(x_v, [b + 1]),
            plsc.load_gather(x_v, [b + 2]))

  def lerp(g, carry, *, hr, lvl, rows_v):
    del carry
    x0, x1, x2 = load_x(g)
    t0 = (x0 + 1.0) * hr
    t1 = (x1 + 1.0) * hr
    t2 = (x2 + 1.0) * hr
    w0 = t0 - t0.astype(jnp.int32).astype(jnp.float32)
    w1 = t1 - t1.astype(jnp.int32).astype(jnp.float32)
    w2 = t2 - t2.astype(jnp.int32).astype(jnp.float32)
    rb = g * 256 + lane
    outs = []
    for f in (0, 1):
      v = [plsc.load_gather(rows_v, [rb + (c * 2 + f) * _LANES])
           for c in range(8)]
      c00 = v[0] + w0 * (v[4] - v[0])
      c01 = v[1] + w0 * (v[5] - v[1])
      c10 = v[2] + w0 * (v[6] - v[2])
      c11 = v[3] + w0 * (v[7] - v[3])
      c0 = c00 + w1 * (c10 - c00)
      c1 = c01 + w1 * (c11 - c01)
      outs.append(c0 + w2 * (c1 - c0))
    pos = g * (_LANES * _OUTW) + ow + 2 * lvl
    plsc.store_scatter(out_v, [pos], outs[0])
    plsc.store_scatter(out_v, [pos + 1], outs[1])
    return 0

  def make_idx_fn(lvl, idx_v):
    hr = float(_RES[lvl]) * 0.5
    if lvl < _N_DENSE:
      ri = int(_RES[lvl])
      ri2 = ri * ri
      ccs = [i * ri2 + j * ri + k for (i, j, k) in _OFFS]

      def idx_dense(g, _):
        x0, x1, x2 = load_x(g)
        b0 = ((x0 + 1.0) * hr).astype(jnp.int32)
        b1 = ((x1 + 1.0) * hr).astype(jnp.int32)
        b2 = ((x2 + 1.0) * hr).astype(jnp.int32)
        ib = g * 256
        bid = b0 * ri2 + b1 * ri + b2
        for c in range(8):
          flat = 2 * (bid + ccs[c])
          idx_v[pl.ds(ib + 2 * c * _LANES, _LANES)] = flat
          idx_v[pl.ds(ib + (2 * c + 1) * _LANES, _LANES)] = flat + 1
        return 0

      return idx_dense

    def idx_hash(g, _):
      x0, x1, x2 = load_x(g)
      b0 = ((x0 + 1.0) * hr).astype(jnp.int32)
      b1 = ((x1 + 1.0) * hr).astype(jnp.int32)
      b2 = ((x2 + 1.0) * hr).astype(jnp.int32)
      ib = g * 256
      m1a = b1 * jnp.int32(_P1)
      m1b = m1a + jnp.int32(_P1)
      m2a = b2 * jnp.int32(_P2)
      m2b = m2a + jnp.int32(_P2)
      b0p = b0 + 1
      for c, (i, j, k) in enumerate(_OFFS):
        h = (b0p if i else b0) ^ (m1b if j else m1a) ^ (m2b if k else m2a)
        flat = 2 * (h & jnp.int32(_MASK))
        idx_v[pl.ds(ib + 2 * c * _LANES, _LANES)] = flat
        idx_v[pl.ds(ib + (2 * c + 1) * _LANES, _LANES)] = flat + 1
      return 0

    return idx_hash

  def chunk_body(ci, _):
    base = wid * ppw + ci * chunk
    pltpu.sync_copy(x_hbm.at[pl.ds(3 * base, 3 * chunk)], x_v)

    lax.fori_loop(0, g16, make_idx_fn(0, idx_bufs[0]), 0)
    h_prev = pltpu.async_copy(tabs[0].at[idx_bufs[0]], rows_bufs[0], sems[0])
    for lvl in range(1, _N_LEVELS):
      b = lvl % 2
      lax.fori_loop(0, g16, make_idx_fn(lvl, idx_bufs[b]), 0)
      h = pltpu.async_copy(tabs[lvl].at[idx_bufs[b]], rows_bufs[b], sems[b])
      h_prev.wait()
      lax.fori_loop(
          0, g16,
          functools.partial(lerp, hr=float(_RES[lvl - 1]) * 0.5,
                            lvl=lvl - 1, rows_v=rows_bufs[1 - b]), 0)
      h_prev = h
    h_prev.wait()
    lax.fori_loop(
        0, g16,
        functools.partial(lerp, hr=float(_RES[_N_LEVELS - 1]) * 0.5,
                          lvl=_N_LEVELS - 1,
                          rows_v=rows_bufs[(_N_LEVELS - 1) % 2]), 0)

    pltpu.sync_copy(out_v, out_hbm.at[pl.ds(base * _OUTW, chunk * _OUTW)])
    return 0

  lax.fori_loop(0, nchunks, chunk_body, 0)


@functools.partial(jax.jit, static_argnames=("interpret",))
def _run(x, tables, interpret=False):
  batch = x.shape[0]
  chunk = min(128, batch // _NW)
  mesh = plsc.VectorSubcoreMesh(
      core_axis_name="c", subcore_axis_name="s",
      num_cores=_NC, num_subcores=_NS)
  body = functools.partial(_sc_body, batch, chunk)
  fn = pl.kernel(
      body,
      out_type=jax.ShapeDtypeStruct((batch * _OUTW,), jnp.float32),
      mesh=mesh,
      scratch_types=[
          pltpu.VMEM((3 * chunk,), jnp.float32),
          pltpu.VMEM((16 * chunk,), jnp.int32),
          pltpu.VMEM((16 * chunk,), jnp.int32),
          pltpu.VMEM((16 * chunk,), jnp.float32),
          pltpu.VMEM((16 * chunk,), jnp.float32),
          pltpu.VMEM((chunk * _OUTW,), jnp.float32),
          pltpu.SemaphoreType.DMA,
          pltpu.SemaphoreType.DMA,
      ],
      compiler_params=pltpu.CompilerParams(needs_layout_passes=False),
      interpret=interpret,
  )
  out = fn(x.reshape(-1), *[t.reshape(-1) for t in tables])
  return out.reshape(batch, _OUTW)


def kernel(x, tables):
  return _run(x, tables)


# speedup vs baseline: 46.9544x; 1.0027x over previous
import functools

import jax
import jax.numpy as jnp
import numpy as np
from jax import lax
from jax.experimental import pallas as pl
from jax.experimental.pallas import tpu as pltpu
from jax.experimental.pallas import tpu_sc as plsc

_N_LEVELS = 16
_F = 2
_LOG2_T = 19
_T = 1 << _LOG2_T
_BASE = np.float32(16.0)
_FINEST = np.float32(512.0)
_B_GROWTH = np.float32(
    np.exp((np.log(_FINEST) - np.log(_BASE)) / np.float32(_N_LEVELS - 1)))
_RES = [np.float32(np.floor(_BASE * (_B_GROWTH ** i))) for i in range(_N_LEVELS)]
_TBL = [int((int(r) + 1) ** 3) if int(r) ** 3 < _T else _T for r in _RES]
_OFFS = [(i, j, k) for i in (0, 1) for j in (0, 1) for k in (0, 1)]
_P1 = int(np.uint32(2654435761).view(np.int32))
_P2 = int(np.uint32(805459861).view(np.int32))
_MASK = _T - 1
_N_DENSE = sum(1 for r in _RES if int(r) ** 3 < _T)

_LANES = 16
_NC = 2
_NS = 16
_NW = _NC * _NS
_OUTW = 2 * _N_LEVELS


def _sc_body(batch, chunk, *refs):
  x_hbm = refs[0]
  tabs = refs[1:1 + _N_LEVELS]
  out_hbm = refs[1 + _N_LEVELS]
  (x_v, idx_v0, idx_v1, rows_v0, rows_v1, out_v,
   sem0, sem1) = refs[2 + _N_LEVELS:]
  idx_bufs = (idx_v0, idx_v1)
  rows_bufs = (rows_v0, rows_v1)
  sems = (sem0, sem1)

  cid = lax.axis_index("c")
  sid = lax.axis_index("s")
  wid = sid * _NC + cid
  ppw = batch // _NW
  nchunks = ppw // chunk
  g16 = chunk // _LANES
  lane = lax.iota(jnp.int32, _LANES)
  x3 = lane * 3
  ow = lane * _OUTW

  def load_x(g):
    b = g * 48 + x3
    return (plsc.load_gather(x_v, [b]),
            plsc.load_gather(x_v, [b + 1]),
            plsc.load_gather(x_v, [b + 2]))

  def lerp(g, carry, *, hr, lvl, rows_v):
    del carry
    x0, x1, x2 = load_x(g)
    t0 = (x0 + 1.0) * hr
    t1 = (x1 + 1.0) * hr
    t2 = (x2 + 1.0) * hr
    w0 = t0 - t0.astype(jnp.int32).astype(jnp.float32)
    w1 = t1 - t1.astype(jnp.int32).astype(jnp.float32)
    w2 = t2 - t2.astype(jnp.int32).astype(jnp.float32)
    rb = g * 256 + lane
    outs = []
    for f in (0, 1):
      v = [plsc.load_gather(rows_v, [rb + (c * 2 + f) * _LANES])
           for c in range(8)]
      c00 = v[0] + w0 * (v[4] - v[0])
      c01 = v[1] + w0 * (v[5] - v[1])
      c10 = v[2] + w0 * (v[6] - v[2])
      c11 = v[3] + w0 * (v[7] - v[3])
      c0 = c00 + w1 * (c10 - c00)
      c1 = c01 + w1 * (c11 - c01)
      outs.append(c0 + w2 * (c1 - c0))
    pos = g * (_LANES * _OUTW) + ow + 2 * lvl
    plsc.store_scatter(out_v, [pos], outs[0])
    plsc.store_scatter(out_v, [pos + 1], outs[1])
    return 0

  def make_idx_fn(lvl, idx_v):
    hr = float(_RES[lvl]) * 0.5
    if lvl < _N_DENSE:
      ri = int(_RES[lvl])
      ri2 = ri * ri
      ccs = [i * ri2 + j * ri + k for (i, j, k) in _OFFS]

      def idx_dense(g, _):
        x0, x1, x2 = load_x(g)
        b0 = ((x0 + 1.0) * hr).astype(jnp.int32)
        b1 = ((x1 + 1.0) * hr).astype(jnp.int32)
        b2 = ((x2 + 1.0) * hr).astype(jnp.int32)
        ib = g * 256
        bid = b0 * ri2 + b1 * ri + b2
        for c in range(8):
          flat = 2 * (bid + ccs[c])
          idx_v[pl.ds(ib + 2 * c * _LANES, _LANES)] = flat
          idx_v[pl.ds(ib + (2 * c + 1) * _LANES, _LANES)] = flat + 1
        return 0

      return idx_dense

    def idx_hash(g, _):
      x0, x1, x2 = load_x(g)
      b0 = ((x0 + 1.0) * hr).astype(jnp.int32)
      b1 = ((x1 + 1.0) * hr).astype(jnp.int32)
      b2 = ((x2 + 1.0) * hr).astype(jnp.int32)
      ib = g * 256
      m1a = b1 * jnp.int32(_P1)
      m1b = m1a + jnp.int32(_P1)
      m2a = b2 * jnp.int32(_P2)
      m2b = m2a + jnp.int32(_P2)
      b0p = b0 + 1
      for c, (i, j, k) in enumerate(_OFFS):
        h = (b0p if i else b0) ^ (m1b if j else m1a) ^ (m2b if k else m2a)
        flat = 2 * (h & jnp.int32(_MASK))
        idx_v[pl.ds(ib + 2 * c * _LANES, _LANES)] = flat
        idx_v[pl.ds(ib + (2 * c + 1) * _LANES, _LANES)] = flat + 1
      return 0

    return idx_hash

  def chunk_body(ci, _):
    base = wid * ppw + ci * chunk
    pltpu.sync_copy(x_hbm.at[pl.ds(3 * base, 3 * chunk)], x_v)

    lax.fori_loop(0, g16, make_idx_fn(0, idx_bufs[0]), 0)
    h_prev = pltpu.async_copy(tabs[0].at[idx_bufs[0]], rows_bufs[0], sems[0])
    for lvl in range(1, _N_LEVELS):
      b = lvl % 2
      lax.fori_loop(0, g16, make_idx_fn(lvl, idx_bufs[b]), 0)
      h = pltpu.async_copy(tabs[lvl].at[idx_bufs[b]], rows_bufs[b], sems[b])
      h_prev.wait()
      lax.fori_loop(
          0, g16,
          functools.partial(lerp, hr=float(_RES[lvl - 1]) * 0.5,
                            lvl=lvl - 1, rows_v=rows_bufs[1 - b]), 0)
      h_prev = h
    h_prev.wait()
    lax.fori_loop(
        0, g16,
        functools.partial(lerp, hr=float(_RES[_N_LEVELS - 1]) * 0.5,
                          lvl=_N_LEVELS - 1,
                          rows_v=rows_bufs[(_N_LEVELS - 1) % 2]), 0)

    pltpu.sync_copy(out_v, out_hbm.at[pl.ds(base * _OUTW, chunk * _OUTW)])
    return 0

  lax.fori_loop(0, nchunks, chunk_body, 0)


@functools.partial(jax.jit, static_argnames=("interpret",))
def _run(x, tables, interpret=False):
  batch = x.shape[0]
  chunk = min(64, batch // _NW)
  mesh = plsc.VectorSubcoreMesh(
      core_axis_name="c", subcore_axis_name="s",
      num_cores=_NC, num_subcores=_NS)
  body = functools.partial(_sc_body, batch, chunk)
  fn = pl.kernel(
      body,
      out_type=jax.ShapeDtypeStruct((batch * _OUTW,), jnp.float32),
      mesh=mesh,
      scratch_types=[
          pltpu.VMEM((3 * chunk,), jnp.float32),
          pltpu.VMEM((16 * chunk,), jnp.int32),
          pltpu.VMEM((16 * chunk,), jnp.int32),
          pltpu.VMEM((16 * chunk,), jnp.float32),
          pltpu.VMEM((16 * chunk,), jnp.float32),
          pltpu.VMEM((chunk * _OUTW,), jnp.float32),
          pltpu.SemaphoreType.DMA,
          pltpu.SemaphoreType.DMA,
      ],
      compiler_params=pltpu.CompilerParams(needs_layout_passes=False),
      interpret=interpret,
  )
  out = fn(x.reshape(-1), *[t.reshape(-1) for t in tables])
  return out.reshape(batch, _OUTW)


def kernel(x, tables):
  return _run(x, tables)
